# baseline (speedup 1.0000x reference)
import jax
import jax.numpy as jnp
from jax import lax
from jax.experimental import pallas as pl
from jax.experimental.pallas import tpu as pltpu
from jax.experimental.pallas import tpu_sc as plsc

NUM_FEATURES = 26
BATCH = 16384
EMBED_DIM = 32
VOCAB = 100001
HALF_B = BATCH // 2
CHUNK = 2048
NCH_FULL = 48
TAIL_MAIN = 1664
TAIL_START = NCH_FULL * CHUNK + TAIL_MAIN
NCH = 50
GROUPS = 512


def _unit_body(idx_ref, tbl_t, tails, out_t, f, g, bh, idx_v, hit_v, hit_p,
               cnt_v, stage, buf0, buf1, sems):
    base_b = bh * HALF_B
    g8 = pl.multiple_of(g * 8, 8)

    pltpu.sync_copy(idx_ref.at[pl.ds(base_b, HALF_B)], idx_v)

    iota = lax.iota(jnp.int32, 16)
    zeros = jnp.zeros((16,), jnp.int32)
    for z in range(4):
        cnt_v[pl.ds(z * 16, 16)] = zeros
    hit_p[pl.ds(HALF_B, 16)] = jnp.full((16,), HALF_B, jnp.int32)

    def count_step(i, carry):
        v = idx_v[pl.ds(i * 16, 16)]
        cid = lax.shift_right_logical(v, 11)
        rank, last = plsc.scan_count(cid)
        base = plsc.load_gather(cnt_v, [cid])
        plsc.store_scatter(cnt_v, [cid], base + rank, mask=last)
        return carry

    lax.fori_loop(0, GROUPS, count_step, 0, unroll=2)

    carry = jnp.int32(0)
    exs = []
    for z in range(4):
        cz = cnt_v[pl.ds(z * 16, 16)]
        inc = plsc.cumsum(cz)
        exs.append(inc - cz + carry)
        carry = carry + jnp.sum(cz, axis=0)
    for z in range(4):
        cnt_v[pl.ds(z * 16, 16)] = exs[z]

    def place_step(i, carry):
        v = idx_v[pl.ds(i * 16, 16)]
        cid = lax.shift_right_logical(v, 11)
        rank, last = plsc.scan_count(cid)
        base = plsc.load_gather(cnt_v, [cid])
        slot = base + rank - 1
        plsc.store_scatter(hit_v, [slot], v)
        plsc.store_scatter(hit_p, [slot], iota + i * 16)
        plsc.store_scatter(cnt_v, [cid], base + rank, mask=last)
        return carry

    lax.fori_loop(0, GROUPS, place_step, 0, unroll=2)

    def cnt_at(k):
        base = lax.div(k, 16) * 16
        vz = cnt_v[pl.ds(base, 16)]
        lane = lax.rem(k, 16)
        sp = jnp.take(vz, jnp.full((16,), lane, jnp.int32), axis=0)
        return lax.squeeze(lax.slice(sp, (0,), (1,)), (0,))

    def extract(vbase, buf, start, end):
        def ex_step(i, carry):
            e0 = start + i * 16
            v = hit_v[pl.ds(e0, 16)]
            col = hit_p[pl.ds(e0, 16)]
            vv = jnp.minimum(jnp.maximum(v - vbase, 0), CHUNK - 1)

            def cstep(c, carry2):
                cs = jnp.full((16,), c, jnp.int32)
                val = plsc.load_gather(buf, [cs, vv])
                plsc.store_scatter(stage, [cs, col], val)
                return carry2

            lax.fori_loop(0, 8, cstep, 0, unroll=False)
            return carry

        n = end - start
        lax.fori_loop(0, lax.div(n + 15, 16), ex_step, 0, unroll=False)

    pltpu.make_async_copy(
        tbl_t.at[pl.ds(g8, 8), pl.ds(0, CHUNK)], buf0, sems.at[0]).start()

    def sweep(kk, start0):
        k0 = kk * 2
        pltpu.make_async_copy(
            tbl_t.at[pl.ds(g8, 8), pl.ds((k0 + 1) * CHUNK, CHUNK)],
            buf1, sems.at[1]).start()
        pltpu.make_async_copy(
            tbl_t.at[pl.ds(g8, 8), pl.ds(k0 * CHUNK, CHUNK)],
            buf0, sems.at[0]).wait()
        end0 = cnt_at(k0)
        extract(k0 * CHUNK, buf0, start0, end0)

        @pl.when(kk < 23)
        def _():
            pltpu.make_async_copy(
                tbl_t.at[pl.ds(g8, 8), pl.ds((k0 + 2) * CHUNK, CHUNK)],
                buf0, sems.at[0]).start()

        pltpu.make_async_copy(
            tbl_t.at[pl.ds(g8, 8), pl.ds((k0 + 1) * CHUNK, CHUNK)],
            buf1, sems.at[1]).wait()
        end1 = cnt_at(k0 + 1)
        extract((k0 + 1) * CHUNK, buf1, end0, end1)
        return end1

    start48 = lax.fori_loop(0, NCH_FULL // 2, sweep, jnp.int32(0),
                            unroll=False)

    pltpu.sync_copy(tbl_t.at[pl.ds(g8, 8), pl.ds(NCH_FULL * CHUNK, TAIL_MAIN)],
                    buf0.at[:, pl.ds(0, TAIL_MAIN)])
    pltpu.sync_copy(tails.at[f, pl.ds(g8, 8), :],
                    buf0.at[:, pl.ds(TAIL_MAIN, 128)])
    extract(NCH_FULL * CHUNK, buf0, start48, jnp.int32(HALF_B))

    pltpu.sync_copy(stage.at[:, pl.ds(0, HALF_B)],
                    out_t.at[pl.ds(pl.multiple_of(32 * f + g8, 8), 8),
                             pl.ds(base_b, HALF_B)])


def _sc_body(*refs):
    idx_refs = refs[:NUM_FEATURES]
    tbl_refs = refs[NUM_FEATURES:2 * NUM_FEATURES]
    tails = refs[2 * NUM_FEATURES]
    out_t = refs[2 * NUM_FEATURES + 1]
    (idx_v, hit_v, hit_p, cnt_v, stage, buf0, buf1,
     sems) = refs[2 * NUM_FEATURES + 2:]

    c = lax.axis_index("c")
    t = lax.axis_index("s")
    s_slot = lax.shift_right_logical(t, 3)
    u = jnp.bitwise_and(t, 7)
    g = lax.shift_right_logical(u, 1)
    bh = jnp.bitwise_and(u, 1)

    for ci in range(2):
        feats = list(range(13)) if ci == 0 else list(range(13, 26))
        for r in range(7):
            for si in range(2):
                pos = 2 * r + si
                if pos >= 13:
                    continue
                f = feats[pos]

                @pl.when(jnp.logical_and(c == ci, s_slot == si))
                def _(f=f):
                    _unit_body(idx_refs[f], tbl_refs[f], tails, out_t, f, g,
                               bh, idx_v, hit_v, hit_p, cnt_v, stage, buf0,
                               buf1, sems)


@jax.jit
def _run(idxs, tbls):
    tails = jnp.stack([
        jnp.pad(t[TAIL_START:, :], ((0, 128 - (VOCAB - TAIL_START)), (0, 0))).T
        for t in tbls
    ])

    sc_call = pl.kernel(
        _sc_body,
        out_type=jax.ShapeDtypeStruct((NUM_FEATURES * EMBED_DIM, BATCH),
                                      jnp.float32),
        mesh=plsc.VectorSubcoreMesh(core_axis_name="c", subcore_axis_name="s"),
        scratch_types=[
            pltpu.VMEM((HALF_B,), jnp.int32),
            pltpu.VMEM((HALF_B + 16,), jnp.int32),
            pltpu.VMEM((HALF_B + 16,), jnp.int32),
            pltpu.VMEM((64,), jnp.int32),
            pltpu.VMEM((8, HALF_B + 16), jnp.float32),
            pltpu.VMEM((8, CHUNK), jnp.float32),
            pltpu.VMEM((8, CHUNK), jnp.float32),
            pltpu.SemaphoreType.DMA((2,)),
        ],
        compiler_params=pltpu.CompilerParams(needs_layout_passes=False),
    )
    out_t = sc_call(*idxs, *[t.T for t in tbls], tails)
    return out_t.T


def kernel(idx_f00, idx_f01, idx_f02, idx_f03, idx_f04, idx_f05, idx_f06,
           idx_f07, idx_f08, idx_f09, idx_f10, idx_f11, idx_f12, idx_f13,
           idx_f14, idx_f15, idx_f16, idx_f17, idx_f18, idx_f19, idx_f20,
           idx_f21, idx_f22, idx_f23, idx_f24, idx_f25, tbl_f00, tbl_f01,
           tbl_f02, tbl_f03, tbl_f04, tbl_f05, tbl_f06, tbl_f07, tbl_f08,
           tbl_f09, tbl_f10, tbl_f11, tbl_f12, tbl_f13, tbl_f14, tbl_f15,
           tbl_f16, tbl_f17, tbl_f18, tbl_f19, tbl_f20, tbl_f21, tbl_f22,
           tbl_f23, tbl_f24, tbl_f25):
    idxs = (idx_f00, idx_f01, idx_f02, idx_f03, idx_f04, idx_f05, idx_f06,
            idx_f07, idx_f08, idx_f09, idx_f10, idx_f11, idx_f12, idx_f13,
            idx_f14, idx_f15, idx_f16, idx_f17, idx_f18, idx_f19, idx_f20,
            idx_f21, idx_f22, idx_f23, idx_f24, idx_f25)
    tbls = (tbl_f00, tbl_f01, tbl_f02, tbl_f03, tbl_f04, tbl_f05, tbl_f06,
            tbl_f07, tbl_f08, tbl_f09, tbl_f10, tbl_f11, tbl_f12, tbl_f13,
            tbl_f14, tbl_f15, tbl_f16, tbl_f17, tbl_f18, tbl_f19, tbl_f20,
            tbl_f21, tbl_f22, tbl_f23, tbl_f24, tbl_f25)
    return _run(idxs, tbls)


# speedup vs baseline: 1.0014x; 1.0014x over previous
import jax
import jax.numpy as jnp
from jax import lax
from jax.experimental import pallas as pl
from jax.experimental.pallas import tpu as pltpu
from jax.experimental.pallas import tpu_sc as plsc

NUM_FEATURES = 26
BATCH = 16384
EMBED_DIM = 32
VOCAB = 100001
HALF_B = BATCH // 2
CHUNK = 2048
NCH_FULL = 48
TAIL_MAIN = 1664
TAIL_START = NCH_FULL * CHUNK + TAIL_MAIN
GROUPS = 512


def _unit_body(idx_ref, tbl_t, tails, out_t, f, g, bh, idx_v, hit_v, hit_p,
               cnt_v, stage, buf0, buf1, sems):
    base_b = bh * HALF_B
    g8 = pl.multiple_of(g * 8, 8)

    pltpu.sync_copy(idx_ref.at[pl.ds(base_b, HALF_B)], idx_v)

    iota = lax.iota(jnp.int32, 16)
    zeros = jnp.zeros((16,), jnp.int32)
    for z in range(4):
        cnt_v[pl.ds(z * 16, 16)] = zeros
    hit_p[pl.ds(HALF_B, 16)] = jnp.full((16,), HALF_B, jnp.int32)

    def count_step(i, carry):
        v = idx_v[pl.ds(i * 16, 16)]
        cid = lax.shift_right_logical(v, 11)
        rank, last = plsc.scan_count(cid)
        base = plsc.load_gather(cnt_v, [cid])
        plsc.store_scatter(cnt_v, [cid], base + rank, mask=last)
        return carry

    lax.fori_loop(0, GROUPS, count_step, 0, unroll=2)

    carry = jnp.int32(0)
    exs = []
    for z in range(4):
        cz = cnt_v[pl.ds(z * 16, 16)]
        inc = plsc.cumsum(cz)
        exs.append(inc - cz + carry)
        carry = carry + jnp.sum(cz, axis=0)
    for z in range(4):
        cnt_v[pl.ds(z * 16, 16)] = exs[z]

    def place_step(i, carry):
        v = idx_v[pl.ds(i * 16, 16)]
        cid = lax.shift_right_logical(v, 11)
        rank, last = plsc.scan_count(cid)
        base = plsc.load_gather(cnt_v, [cid])
        slot = base + rank - 1
        plsc.store_scatter(hit_v, [slot], v)
        plsc.store_scatter(hit_p, [slot], iota + i * 16)
        plsc.store_scatter(cnt_v, [cid], base + rank, mask=last)
        return carry

    lax.fori_loop(0, GROUPS, place_step, 0, unroll=2)

    def cnt_at(k):
        base = lax.div(k, 16) * 16
        vz = cnt_v[pl.ds(base, 16)]
        lane = lax.rem(k, 16)
        sp = jnp.take(vz, jnp.full((16,), lane, jnp.int32), axis=0)
        return lax.squeeze(lax.slice(sp, (0,), (1,)), (0,))

    def extract(vbase, buf, start, end):
        def ex_step(i, carry):
            e0 = start + i * 16
            v = hit_v[pl.ds(e0, 16)]
            col = hit_p[pl.ds(e0, 16)]
            vv = jnp.minimum(jnp.maximum(v - vbase, 0), CHUNK - 1)

            def cstep(c, carry2):
                cs = jnp.full((16,), c, jnp.int32)
                val = plsc.load_gather(buf, [cs, vv])
                plsc.store_scatter(stage, [cs, col], val)
                return carry2

            lax.fori_loop(0, 8, cstep, 0, unroll=False)
            return carry

        n = end - start
        lax.fori_loop(0, lax.div(n + 15, 16), ex_step, 0, unroll=False)

    pltpu.make_async_copy(
        tbl_t.at[pl.ds(g8, 8), pl.ds(0, CHUNK)], buf0, sems.at[0]).start()

    def sweep(kk, start0):
        k0 = kk * 2
        pltpu.make_async_copy(
            tbl_t.at[pl.ds(g8, 8), pl.ds((k0 + 1) * CHUNK, CHUNK)],
            buf1, sems.at[1]).start()
        pltpu.make_async_copy(
            tbl_t.at[pl.ds(g8, 8), pl.ds(k0 * CHUNK, CHUNK)],
            buf0, sems.at[0]).wait()
        end0 = cnt_at(k0)
        extract(k0 * CHUNK, buf0, start0, end0)

        @pl.when(kk < 23)
        def _():
            pltpu.make_async_copy(
                tbl_t.at[pl.ds(g8, 8), pl.ds((k0 + 2) * CHUNK, CHUNK)],
                buf0, sems.at[0]).start()

        pltpu.make_async_copy(
            tbl_t.at[pl.ds(g8, 8), pl.ds((k0 + 1) * CHUNK, CHUNK)],
            buf1, sems.at[1]).wait()
        end1 = cnt_at(k0 + 1)
        extract((k0 + 1) * CHUNK, buf1, end0, end1)
        return end1

    start48 = lax.fori_loop(0, NCH_FULL // 2, sweep, jnp.int32(0),
                            unroll=False)

    pltpu.sync_copy(tbl_t.at[pl.ds(g8, 8), pl.ds(NCH_FULL * CHUNK, TAIL_MAIN)],
                    buf0.at[:, pl.ds(0, TAIL_MAIN)])
    pltpu.sync_copy(tails.at[f, pl.ds(g8, 8), :],
                    buf0.at[:, pl.ds(TAIL_MAIN, 128)])
    extract(NCH_FULL * CHUNK, buf0, start48, jnp.int32(HALF_B))

    pltpu.sync_copy(stage.at[:, pl.ds(0, HALF_B)],
                    out_t.at[pl.ds(pl.multiple_of(32 * f + g8, 8), 8),
                             pl.ds(base_b, HALF_B)])


def _sc_body(*refs):
    idx_refs = refs[:NUM_FEATURES]
    tbl_refs = refs[NUM_FEATURES:2 * NUM_FEATURES]
    tails = refs[2 * NUM_FEATURES]
    out_t = refs[2 * NUM_FEATURES + 1]
    (idx_v, hit_v, hit_p, cnt_v, stage, buf0, buf1,
     sems) = refs[2 * NUM_FEATURES + 2:]

    c = lax.axis_index("c")
    t = lax.axis_index("s")
    s_slot = lax.shift_right_logical(t, 3)
    u = jnp.bitwise_and(t, 7)
    g = lax.shift_right_logical(u, 1)
    bh = jnp.bitwise_and(u, 1)

    for ci in range(2):
        feats = list(range(13)) if ci == 0 else list(range(13, 26))
        for r in range(7):
            for si in range(2):
                pos = 2 * r + si
                if pos >= 13:
                    continue
                f = feats[pos]

                @pl.when(jnp.logical_and(c == ci, s_slot == si))
                def _(f=f):
                    _unit_body(idx_refs[f], tbl_refs[f], tails, out_t, f, g,
                               bh, idx_v, hit_v, hit_p, cnt_v, stage, buf0,
                               buf1, sems)


@jax.jit
def _run(idxs, tbls):
    tails = jnp.stack([
        jnp.pad(t[TAIL_START:, :], ((0, 128 - (VOCAB - TAIL_START)), (0, 0))).T
        for t in tbls
    ])

    sc_call = pl.kernel(
        _sc_body,
        out_type=jax.ShapeDtypeStruct((NUM_FEATURES * EMBED_DIM, BATCH),
                                      jnp.float32),
        mesh=plsc.VectorSubcoreMesh(core_axis_name="c", subcore_axis_name="s"),
        scratch_types=[
            pltpu.VMEM((HALF_B,), jnp.int32),
            pltpu.VMEM((HALF_B + 16,), jnp.int32),
            pltpu.VMEM((HALF_B + 16,), jnp.int32),
            pltpu.VMEM((64,), jnp.int32),
            pltpu.VMEM((8, HALF_B + 16), jnp.float32),
            pltpu.VMEM((8, CHUNK), jnp.float32),
            pltpu.VMEM((8, CHUNK), jnp.float32),
            pltpu.SemaphoreType.DMA((2,)),
        ],
        compiler_params=pltpu.CompilerParams(needs_layout_passes=False),
    )
    out_t = sc_call(*idxs, *[t.T for t in tbls], tails)
    return out_t.T


def kernel(idx_f00, idx_f01, idx_f02, idx_f03, idx_f04, idx_f05, idx_f06,
           idx_f07, idx_f08, idx_f09, idx_f10, idx_f11, idx_f12, idx_f13,
           idx_f14, idx_f15, idx_f16, idx_f17, idx_f18, idx_f19, idx_f20,
           idx_f21, idx_f22, idx_f23, idx_f24, idx_f25, tbl_f00, tbl_f01,
           tbl_f02, tbl_f03, tbl_f04, tbl_f05, tbl_f06, tbl_f07, tbl_f08,
           tbl_f09, tbl_f10, tbl_f11, tbl_f12, tbl_f13, tbl_f14, tbl_f15,
           tbl_f16, tbl_f17, tbl_f18, tbl_f19, tbl_f20, tbl_f21, tbl_f22,
           tbl_f23, tbl_f24, tbl_f25):
    idxs = (idx_f00, idx_f01, idx_f02, idx_f03, idx_f04, idx_f05, idx_f06,
            idx_f07, idx_f08, idx_f09, idx_f10, idx_f11, idx_f12, idx_f13,
            idx_f14, idx_f15, idx_f16, idx_f17, idx_f18, idx_f19, idx_f20,
            idx_f21, idx_f22, idx_f23, idx_f24, idx_f25)
    tbls = (tbl_f00, tbl_f01, tbl_f02, tbl_f03, tbl_f04, tbl_f05, tbl_f06,
            tbl_f07, tbl_f08, tbl_f09, tbl_f10, tbl_f11, tbl_f12, tbl_f13,
            tbl_f14, tbl_f15, tbl_f16, tbl_f17, tbl_f18, tbl_f19, tbl_f20,
            tbl_f21, tbl_f22, tbl_f23, tbl_f24, tbl_f25)
    return _run(idxs, tbls)


# speedup vs baseline: 1.0273x; 1.0259x over previous
import jax
import jax.numpy as jnp
from jax import lax
from jax.experimental import pallas as pl
from jax.experimental.pallas import tpu as pltpu
from jax.experimental.pallas import tpu_sc as plsc

NUM_FEATURES = 26
BATCH = 16384
EMBED_DIM = 32
VOCAB = 100001
HALF_B = BATCH // 2
CHUNK = 2048
NCH_FULL = 48
TAIL_MAIN = 1664
TAIL_START = NCH_FULL * CHUNK + TAIL_MAIN
GROUPS = 512


def _unit_body(idx_ref, tbl_t, tails, out_t, f, g, bh, idx_v, hit_v, hit_p,
               cnt_v, stage, buf0, buf1, sems, out_sem, is_first):
    base_b = bh * HALF_B
    g8 = pl.multiple_of(g * 8, 8)

    pltpu.sync_copy(idx_ref.at[pl.ds(base_b, HALF_B)], idx_v)

    iota = lax.iota(jnp.int32, 16)
    zeros = jnp.zeros((16,), jnp.int32)
    for z in range(4):
        cnt_v[pl.ds(z * 16, 16)] = zeros
    hit_p[pl.ds(HALF_B, 16)] = jnp.full((16,), HALF_B, jnp.int32)

    def count_step(i, carry):
        v = idx_v[pl.ds(i * 16, 16)]
        cid = lax.shift_right_logical(v, 11)
        rank, last = plsc.scan_count(cid)
        base = plsc.load_gather(cnt_v, [cid])
        plsc.store_scatter(cnt_v, [cid], base + rank, mask=last)
        return carry

    lax.fori_loop(0, GROUPS, count_step, 0, unroll=2)

    carry = jnp.int32(0)
    exs = []
    for z in range(4):
        cz = cnt_v[pl.ds(z * 16, 16)]
        inc = plsc.cumsum(cz)
        exs.append(inc - cz + carry)
        carry = carry + jnp.sum(cz, axis=0)
    for z in range(4):
        cnt_v[pl.ds(z * 16, 16)] = exs[z]

    def place_step(i, carry):
        v = idx_v[pl.ds(i * 16, 16)]
        cid = lax.shift_right_logical(v, 11)
        rank, last = plsc.scan_count(cid)
        base = plsc.load_gather(cnt_v, [cid])
        slot = base + rank - 1
        plsc.store_scatter(hit_v, [slot], v)
        plsc.store_scatter(hit_p, [slot], iota + i * 16)
        plsc.store_scatter(cnt_v, [cid], base + rank, mask=last)
        return carry

    lax.fori_loop(0, GROUPS, place_step, 0, unroll=2)

    def cnt_at(k):
        base = lax.div(k, 16) * 16
        vz = cnt_v[pl.ds(base, 16)]
        lane = lax.rem(k, 16)
        sp = jnp.take(vz, jnp.full((16,), lane, jnp.int32), axis=0)
        return lax.squeeze(lax.slice(sp, (0,), (1,)), (0,))

    def extract(vbase, buf, start, end):
        def ex_step(i, carry):
            e0 = start + i * 16
            v = hit_v[pl.ds(e0, 16)]
            col = hit_p[pl.ds(e0, 16)]
            vv = jnp.minimum(jnp.maximum(v - vbase, 0), CHUNK - 1)

            def cstep(c, carry2):
                cs = jnp.full((16,), c, jnp.int32)
                val = plsc.load_gather(buf, [cs, vv])
                plsc.store_scatter(stage, [cs, col], val)
                return carry2

            lax.fori_loop(0, 8, cstep, 0, unroll=False)
            return carry

        n = end - start
        lax.fori_loop(0, lax.div(n + 15, 16), ex_step, 0, unroll=False)

    if not is_first:
        pltpu.make_async_copy(
            stage.at[:, pl.ds(0, HALF_B)],
            out_t.at[pl.ds(pl.multiple_of(32 * f + g8, 8), 8),
                     pl.ds(base_b, HALF_B)], out_sem).wait()

    pltpu.make_async_copy(
        tbl_t.at[pl.ds(g8, 8), pl.ds(0, CHUNK)], buf0, sems.at[0]).start()

    def sweep(kk, start0):
        k0 = kk * 2
        pltpu.make_async_copy(
            tbl_t.at[pl.ds(g8, 8), pl.ds((k0 + 1) * CHUNK, CHUNK)],
            buf1, sems.at[1]).start()
        pltpu.make_async_copy(
            tbl_t.at[pl.ds(g8, 8), pl.ds(k0 * CHUNK, CHUNK)],
            buf0, sems.at[0]).wait()
        end0 = cnt_at(k0)
        extract(k0 * CHUNK, buf0, start0, end0)

        @pl.when(kk < 23)
        def _():
            pltpu.make_async_copy(
                tbl_t.at[pl.ds(g8, 8), pl.ds((k0 + 2) * CHUNK, CHUNK)],
                buf0, sems.at[0]).start()

        pltpu.make_async_copy(
            tbl_t.at[pl.ds(g8, 8), pl.ds((k0 + 1) * CHUNK, CHUNK)],
            buf1, sems.at[1]).wait()
        end1 = cnt_at(k0 + 1)
        extract((k0 + 1) * CHUNK, buf1, end0, end1)
        return end1

    start48 = lax.fori_loop(0, NCH_FULL // 2, sweep, jnp.int32(0),
                            unroll=False)

    pltpu.sync_copy(tbl_t.at[pl.ds(g8, 8), pl.ds(NCH_FULL * CHUNK, TAIL_MAIN)],
                    buf0.at[:, pl.ds(0, TAIL_MAIN)])
    pltpu.sync_copy(tails.at[f, pl.ds(g8, 8), :],
                    buf0.at[:, pl.ds(TAIL_MAIN, 128)])
    extract(NCH_FULL * CHUNK, buf0, start48, jnp.int32(HALF_B))

    pltpu.make_async_copy(
        stage.at[:, pl.ds(0, HALF_B)],
        out_t.at[pl.ds(pl.multiple_of(32 * f + g8, 8), 8),
                 pl.ds(base_b, HALF_B)], out_sem).start()


def _sc_body(*refs):
    idx_refs = refs[:NUM_FEATURES]
    tbl_refs = refs[NUM_FEATURES:2 * NUM_FEATURES]
    tails = refs[2 * NUM_FEATURES]
    out_t = refs[2 * NUM_FEATURES + 1]
    (idx_v, hit_v, hit_p, cnt_v, stage, buf0, buf1,
     sems, out_sem) = refs[2 * NUM_FEATURES + 2:]

    c = lax.axis_index("c")
    t = lax.axis_index("s")
    s_slot = lax.shift_right_logical(t, 3)
    u = jnp.bitwise_and(t, 7)
    g = lax.shift_right_logical(u, 1)
    bh = jnp.bitwise_and(u, 1)

    for ci in range(2):
        feats = list(range(13)) if ci == 0 else list(range(13, 26))
        for r in range(7):
            for si in range(2):
                pos = 2 * r + si
                if pos >= 13:
                    continue
                f = feats[pos]

                @pl.when(jnp.logical_and(c == ci, s_slot == si))
                def _(f=f, r=r):
                    _unit_body(idx_refs[f], tbl_refs[f], tails, out_t, f, g,
                               bh, idx_v, hit_v, hit_p, cnt_v, stage, buf0,
                               buf1, sems, out_sem, r == 0)

    pltpu.make_async_copy(
        stage.at[:, pl.ds(0, HALF_B)],
        out_t.at[pl.ds(0, 8), pl.ds(0, HALF_B)], out_sem).wait()


@jax.jit
def _run(idxs, tbls):
    tails = jnp.stack([
        jnp.pad(t[TAIL_START:, :], ((0, 128 - (VOCAB - TAIL_START)), (0, 0))).T
        for t in tbls
    ])

    sc_call = pl.kernel(
        _sc_body,
        out_type=jax.ShapeDtypeStruct((NUM_FEATURES * EMBED_DIM, BATCH),
                                      jnp.float32),
        mesh=plsc.VectorSubcoreMesh(core_axis_name="c", subcore_axis_name="s"),
        scratch_types=[
            pltpu.VMEM((HALF_B,), jnp.int32),
            pltpu.VMEM((HALF_B + 16,), jnp.int32),
            pltpu.VMEM((HALF_B + 16,), jnp.int32),
            pltpu.VMEM((64,), jnp.int32),
            pltpu.VMEM((8, HALF_B + 16), jnp.float32),
            pltpu.VMEM((8, CHUNK), jnp.float32),
            pltpu.VMEM((8, CHUNK), jnp.float32),
            pltpu.SemaphoreType.DMA((2,)),
            pltpu.SemaphoreType.DMA,
        ],
        compiler_params=pltpu.CompilerParams(needs_layout_passes=False),
    )
    out_t = sc_call(*idxs, *[t.T for t in tbls], tails)
    return out_t.T


def kernel(idx_f00, idx_f01, idx_f02, idx_f03, idx_f04, idx_f05, idx_f06,
           idx_f07, idx_f08, idx_f09, idx_f10, idx_f11, idx_f12, idx_f13,
           idx_f14, idx_f15, idx_f16, idx_f17, idx_f18, idx_f19, idx_f20,
           idx_f21, idx_f22, idx_f23, idx_f24, idx_f25, tbl_f00, tbl_f01,
           tbl_f02, tbl_f03, tbl_f04, tbl_f05, tbl_f06, tbl_f07, tbl_f08,
           tbl_f09, tbl_f10, tbl_f11, tbl_f12, tbl_f13, tbl_f14, tbl_f15,
           tbl_f16, tbl_f17, tbl_f18, tbl_f19, tbl_f20, tbl_f21, tbl_f22,
           tbl_f23, tbl_f24, tbl_f25):
    idxs = (idx_f00, idx_f01, idx_f02, idx_f03, idx_f04, idx_f05, idx_f06,
            idx_f07, idx_f08, idx_f09, idx_f10, idx_f11, idx_f12, idx_f13,
            idx_f14, idx_f15, idx_f16, idx_f17, idx_f18, idx_f19, idx_f20,
            idx_f21, idx_f22, idx_f23, idx_f24, idx_f25)
    tbls = (tbl_f00, tbl_f01, tbl_f02, tbl_f03, tbl_f04, tbl_f05, tbl_f06,
            tbl_f07, tbl_f08, tbl_f09, tbl_f10, tbl_f11, tbl_f12, tbl_f13,
            tbl_f14, tbl_f15, tbl_f16, tbl_f17, tbl_f18, tbl_f19, tbl_f20,
            tbl_f21, tbl_f22, tbl_f23, tbl_f24, tbl_f25)
    return _run(idxs, tbls)


# speedup vs baseline: 1.0485x; 1.0206x over previous
import jax
import jax.numpy as jnp
from jax import lax
from jax.experimental import pallas as pl
from jax.experimental.pallas import tpu as pltpu
from jax.experimental.pallas import tpu_sc as plsc

NUM_FEATURES = 26
BATCH = 16384
EMBED_DIM = 32
VOCAB = 100001
HALF_B = BATCH // 2
CHUNK = 2048
NCH_FULL = 48
TAIL_MAIN = 1664
TAIL_START = NCH_FULL * CHUNK + TAIL_MAIN
GROUPS = 512


def _unit_body(idx_ref, tbl_t, tails, out_t, f, g, bh, idx_v, hit_v, hit_p,
               cnt_v, stage, buf0, buf1, sems, out_sem, is_first):
    base_b = bh * HALF_B
    g8 = pl.multiple_of(g * 8, 8)

    pltpu.sync_copy(idx_ref.at[pl.ds(base_b, HALF_B)], idx_v)

    pltpu.make_async_copy(
        tbl_t.at[pl.ds(g8, 8), pl.ds(0, CHUNK)], buf0, sems.at[0]).start()
    pltpu.make_async_copy(
        tbl_t.at[pl.ds(g8, 8), pl.ds(CHUNK, CHUNK)], buf1, sems.at[1]).start()

    iota = lax.iota(jnp.int32, 16)
    zeros = jnp.zeros((16,), jnp.int32)
    for z in range(4):
        cnt_v[pl.ds(z * 16, 16)] = zeros
    hit_p[pl.ds(HALF_B, 16)] = jnp.full((16,), HALF_B, jnp.int32)

    def count_step(i, carry):
        v = idx_v[pl.ds(i * 16, 16)]
        cid = lax.shift_right_logical(v, 11)
        rank, last = plsc.scan_count(cid)
        base = plsc.load_gather(cnt_v, [cid])
        plsc.store_scatter(cnt_v, [cid], base + rank, mask=last)
        return carry

    lax.fori_loop(0, GROUPS, count_step, 0, unroll=2)

    carry = jnp.int32(0)
    exs = []
    for z in range(4):
        cz = cnt_v[pl.ds(z * 16, 16)]
        inc = plsc.cumsum(cz)
        exs.append(inc - cz + carry)
        carry = carry + jnp.sum(cz, axis=0)
    for z in range(4):
        cnt_v[pl.ds(z * 16, 16)] = exs[z]

    def place_step(i, carry):
        v = idx_v[pl.ds(i * 16, 16)]
        cid = lax.shift_right_logical(v, 11)
        rank, last = plsc.scan_count(cid)
        base = plsc.load_gather(cnt_v, [cid])
        slot = base + rank - 1
        plsc.store_scatter(hit_v, [slot], v)
        plsc.store_scatter(hit_p, [slot], iota + i * 16)
        plsc.store_scatter(cnt_v, [cid], base + rank, mask=last)
        return carry

    lax.fori_loop(0, GROUPS, place_step, 0, unroll=2)

    def cnt_at(k):
        base = lax.div(k, 16) * 16
        vz = cnt_v[pl.ds(base, 16)]
        lane = lax.rem(k, 16)
        sp = jnp.take(vz, jnp.full((16,), lane, jnp.int32), axis=0)
        return lax.squeeze(lax.slice(sp, (0,), (1,)), (0,))

    def extract(vbase, buf, start, end):
        def ex_step(i, carry):
            e0 = start + i * 16
            v = hit_v[pl.ds(e0, 16)]
            col = hit_p[pl.ds(e0, 16)]
            vv = jnp.minimum(jnp.maximum(v - vbase, 0), CHUNK - 1)

            def cstep(c, carry2):
                cs = jnp.full((16,), c, jnp.int32)
                val = plsc.load_gather(buf, [cs, vv])
                plsc.store_scatter(stage, [cs, col], val)
                return carry2

            lax.fori_loop(0, 8, cstep, 0, unroll=False)
            return carry

        n = end - start
        lax.fori_loop(0, lax.div(n + 15, 16), ex_step, 0, unroll=False)

    if not is_first:
        pltpu.make_async_copy(
            stage.at[:, pl.ds(0, HALF_B)],
            out_t.at[pl.ds(pl.multiple_of(32 * f + g8, 8), 8),
                     pl.ds(base_b, HALF_B)], out_sem).wait()

    def sweep(kk, start0):
        k0 = kk * 2
        pltpu.make_async_copy(
            tbl_t.at[pl.ds(g8, 8), pl.ds(k0 * CHUNK, CHUNK)],
            buf0, sems.at[0]).wait()
        end0 = cnt_at(k0)
        extract(k0 * CHUNK, buf0, start0, end0)

        @pl.when(kk < 23)
        def _():
            pltpu.make_async_copy(
                tbl_t.at[pl.ds(g8, 8), pl.ds((k0 + 2) * CHUNK, CHUNK)],
                buf0, sems.at[0]).start()

        pltpu.make_async_copy(
            tbl_t.at[pl.ds(g8, 8), pl.ds((k0 + 1) * CHUNK, CHUNK)],
            buf1, sems.at[1]).wait()
        end1 = cnt_at(k0 + 1)
        extract((k0 + 1) * CHUNK, buf1, end0, end1)

        @pl.when(kk < 23)
        def _():
            pltpu.make_async_copy(
                tbl_t.at[pl.ds(g8, 8), pl.ds((k0 + 3) * CHUNK, CHUNK)],
                buf1, sems.at[1]).start()
        return end1

    start48 = lax.fori_loop(0, NCH_FULL // 2, sweep, jnp.int32(0),
                            unroll=False)

    pltpu.sync_copy(tbl_t.at[pl.ds(g8, 8), pl.ds(NCH_FULL * CHUNK, TAIL_MAIN)],
                    buf0.at[:, pl.ds(0, TAIL_MAIN)])
    pltpu.sync_copy(tails.at[f, pl.ds(g8, 8), :],
                    buf0.at[:, pl.ds(TAIL_MAIN, 128)])
    extract(NCH_FULL * CHUNK, buf0, start48, jnp.int32(HALF_B))

    pltpu.make_async_copy(
        stage.at[:, pl.ds(0, HALF_B)],
        out_t.at[pl.ds(pl.multiple_of(32 * f + g8, 8), 8),
                 pl.ds(base_b, HALF_B)], out_sem).start()


def _sc_body(*refs):
    idx_refs = refs[:NUM_FEATURES]
    tbl_refs = refs[NUM_FEATURES:2 * NUM_FEATURES]
    tails = refs[2 * NUM_FEATURES]
    out_t = refs[2 * NUM_FEATURES + 1]
    (idx_v, hit_v, hit_p, cnt_v, stage, buf0, buf1,
     sems, out_sem) = refs[2 * NUM_FEATURES + 2:]

    c = lax.axis_index("c")
    t = lax.axis_index("s")
    s_slot = lax.shift_right_logical(t, 3)
    u = jnp.bitwise_and(t, 7)
    g = lax.shift_right_logical(u, 1)
    bh = jnp.bitwise_and(u, 1)

    for ci in range(2):
        feats = list(range(13)) if ci == 0 else list(range(13, 26))
        for r in range(7):
            for si in range(2):
                pos = 2 * r + si
                if pos >= 13:
                    continue
                f = feats[pos]

                @pl.when(jnp.logical_and(c == ci, s_slot == si))
                def _(f=f, r=r):
                    _unit_body(idx_refs[f], tbl_refs[f], tails, out_t, f, g,
                               bh, idx_v, hit_v, hit_p, cnt_v, stage, buf0,
                               buf1, sems, out_sem, r == 0)

    pltpu.make_async_copy(
        stage.at[:, pl.ds(0, HALF_B)],
        out_t.at[pl.ds(0, 8), pl.ds(0, HALF_B)], out_sem).wait()


@jax.jit
def _run(idxs, tbls):
    tails = jnp.stack([
        jnp.pad(t[TAIL_START:, :], ((0, 128 - (VOCAB - TAIL_START)), (0, 0))).T
        for t in tbls
    ])

    sc_call = pl.kernel(
        _sc_body,
        out_type=jax.ShapeDtypeStruct((NUM_FEATURES * EMBED_DIM, BATCH),
                                      jnp.float32),
        mesh=plsc.VectorSubcoreMesh(core_axis_name="c", subcore_axis_name="s"),
        scratch_types=[
            pltpu.VMEM((HALF_B,), jnp.int32),
            pltpu.VMEM((HALF_B + 16,), jnp.int32),
            pltpu.VMEM((HALF_B + 16,), jnp.int32),
            pltpu.VMEM((64,), jnp.int32),
            pltpu.VMEM((8, HALF_B + 16), jnp.float32),
            pltpu.VMEM((8, CHUNK), jnp.float32),
            pltpu.VMEM((8, CHUNK), jnp.float32),
            pltpu.SemaphoreType.DMA((2,)),
            pltpu.SemaphoreType.DMA,
        ],
        compiler_params=pltpu.CompilerParams(needs_layout_passes=False),
    )
    out_t = sc_call(*idxs, *[t.T for t in tbls], tails)
    return out_t.T


def kernel(idx_f00, idx_f01, idx_f02, idx_f03, idx_f04, idx_f05, idx_f06,
           idx_f07, idx_f08, idx_f09, idx_f10, idx_f11, idx_f12, idx_f13,
           idx_f14, idx_f15, idx_f16, idx_f17, idx_f18, idx_f19, idx_f20,
           idx_f21, idx_f22, idx_f23, idx_f24, idx_f25, tbl_f00, tbl_f01,
           tbl_f02, tbl_f03, tbl_f04, tbl_f05, tbl_f06, tbl_f07, tbl_f08,
           tbl_f09, tbl_f10, tbl_f11, tbl_f12, tbl_f13, tbl_f14, tbl_f15,
           tbl_f16, tbl_f17, tbl_f18, tbl_f19, tbl_f20, tbl_f21, tbl_f22,
           tbl_f23, tbl_f24, tbl_f25):
    idxs = (idx_f00, idx_f01, idx_f02, idx_f03, idx_f04, idx_f05, idx_f06,
            idx_f07, idx_f08, idx_f09, idx_f10, idx_f11, idx_f12, idx_f13,
            idx_f14, idx_f15, idx_f16, idx_f17, idx_f18, idx_f19, idx_f20,
            idx_f21, idx_f22, idx_f23, idx_f24, idx_f25)
    tbls = (tbl_f00, tbl_f01, tbl_f02, tbl_f03, tbl_f04, tbl_f05, tbl_f06,
            tbl_f07, tbl_f08, tbl_f09, tbl_f10, tbl_f11, tbl_f12, tbl_f13,
            tbl_f14, tbl_f15, tbl_f16, tbl_f17, tbl_f18, tbl_f19, tbl_f20,
            tbl_f21, tbl_f22, tbl_f23, tbl_f24, tbl_f25)
    return _run(idxs, tbls)
